# baseline (speedup 1.0000x reference)
import functools

import jax
import jax.numpy as jnp
from jax import lax
from jax.experimental import pallas as pl
from jax.experimental.pallas import tpu as pltpu
from jax.experimental.pallas import tpu_sc as plsc

BATCH = 16384
EMBED_DIM = 64
L = 16
NC, NS = 2, 16
NW = NC * NS
BPW = BATCH // NW
CH = 128
NCH = BPW // CH
GROUPS = BPW // L


def _mf_body(uidx_hbm, iidx_hbm, utab_hbm, itab_hbm, ub_hbm, ib_hbm, gb_hbm,
             out_hbm,
             uidx_v, iidx_v, urows_v, irows_v, ubv, ibv, gbv, out_v, sem):
    wid = lax.axis_index("s") * NC + lax.axis_index("c")
    base = wid * BPW

    pltpu.sync_copy(uidx_hbm.at[pl.ds(base, BPW)], uidx_v)
    pltpu.sync_copy(iidx_hbm.at[pl.ds(base, BPW)], iidx_v)
    pltpu.sync_copy(gb_hbm, gbv)

    copies = []
    for j in range(NCH):
        s = pl.ds(j * CH, CH)
        copies.append(pltpu.async_copy(utab_hbm.at[uidx_v.at[s]], urows_v.at[s], sem))
        copies.append(pltpu.async_copy(itab_hbm.at[iidx_v.at[s]], irows_v.at[s], sem))
        copies.append(pltpu.async_copy(ub_hbm.at[uidx_v.at[s]], ubv.at[s], sem))
        copies.append(pltpu.async_copy(ib_hbm.at[iidx_v.at[s]], ibv.at[s], sem))
    for c in copies:
        c.wait()

    gb = gbv[...]

    def group(g, carry):
        rbase = g * L
        rows = rbase + lax.iota(jnp.int32, L)
        acc = ubv[pl.ds(rbase, L)] + ibv[pl.ds(rbase, L)] + gb
        dvec = jnp.zeros((L,), jnp.int32)
        for _ in range(EMBED_DIM):
            du = plsc.load_gather(urows_v, [rows, dvec])
            di = plsc.load_gather(irows_v, [rows, dvec])
            acc = acc + du * di
            dvec = dvec + 1
        out_v[pl.ds(rbase, L)] = acc
        return carry

    lax.fori_loop(0, GROUPS, group, 0)

    pltpu.sync_copy(out_v, out_hbm.at[pl.ds(base, BPW)])


def kernel(user_indices, item_indices, user_embedding, item_embedding,
           user_bias, item_bias, global_bias):
    mesh = plsc.VectorSubcoreMesh(core_axis_name="c", subcore_axis_name="s")
    k = pl.kernel(
        _mf_body,
        mesh=mesh,
        compiler_params=pltpu.CompilerParams(needs_layout_passes=False,
                                             use_tc_tiling_on_sc=False),
        out_type=jax.ShapeDtypeStruct((BATCH,), jnp.float32),
        scratch_types=[
            pltpu.VMEM((BPW,), jnp.int32),
            pltpu.VMEM((BPW,), jnp.int32),
            pltpu.VMEM((BPW, EMBED_DIM), jnp.float32),
            pltpu.VMEM((BPW, EMBED_DIM), jnp.float32),
            pltpu.VMEM((BPW,), jnp.float32),
            pltpu.VMEM((BPW,), jnp.float32),
            pltpu.VMEM((L,), jnp.float32),
            pltpu.VMEM((BPW,), jnp.float32),
            pltpu.SemaphoreType.DMA,
        ],
    )
    return k(user_indices.astype(jnp.int32), item_indices.astype(jnp.int32),
             user_embedding, item_embedding,
             user_bias.reshape(-1), item_bias.reshape(-1),
             jnp.broadcast_to(global_bias, (L,)))
